# baseline (speedup 1.0000x reference)
import functools

import jax
import jax.numpy as jnp
from jax import lax
from jax.experimental import pallas as pl
from jax.experimental.pallas import tpu as pltpu
from jax.experimental.pallas import tpu_sc as plsc


def _enc_score_body(x_ref, w1_ref, b1_ref, w2_ref, b2_ref, emb_ref,
                    enc_ref, ind_ref, oh_ref, vqp_ref, embn_ref, prev_ref):
    i = pl.program_id(0)
    nt = pl.num_programs(0) - 1

    @pl.when(i == 0)
    def _norm_codebook():
        v = emb_ref[...]
        n = jnp.sqrt(jnp.sum(v * v, axis=1, keepdims=True)) + 1e-12
        embn_ref[...] = v / n

    @pl.when(i > 0)
    def _prev_tile():
        ip = prev_ref[...]
        iota = lax.broadcasted_iota(jnp.int32, oh_ref.shape, 1)
        oh = (iota == ip).astype(jnp.float32)
        oh_ref[...] = oh
        vqp_ref[...] = jnp.dot(oh, emb_ref[...])

    @pl.when(i < nt)
    def _cur_tile():
        h = jnp.maximum(jnp.dot(x_ref[...], w1_ref[...]) + b1_ref[...], 0.0)
        a = jnp.dot(h, w2_ref[...]) + b2_ref[...]
        enc_ref[...] = a
        an = a / (jnp.sqrt(jnp.sum(a * a, axis=1, keepdims=True)) + 1e-12)
        d = -jnp.dot(an, embn_ref[...].T)
        ind = jnp.argmin(d, axis=1).astype(jnp.int32).reshape(-1, 1)
        ind_ref[...] = ind
        prev_ref[...] = ind


def _encode_and_nearest(x, w1, b1, w2, b2, emb, bt):
    b, f = x.shape
    m = w1.shape[1]
    e = w2.shape[1]
    k = emb.shape[0]
    nt = b // bt

    def cur(i):
        return (jnp.minimum(i, nt - 1), 0)

    def prev(i):
        return (jnp.maximum(i - 1, 0), 0)

    return pl.pallas_call(
        _enc_score_body,
        grid=(nt + 1,),
        in_specs=[
            pl.BlockSpec((bt, f), cur),
            pl.BlockSpec((f, m), lambda i: (0, 0)),
            pl.BlockSpec((1, m), lambda i: (0, 0)),
            pl.BlockSpec((m, e), lambda i: (0, 0)),
            pl.BlockSpec((1, e), lambda i: (0, 0)),
            pl.BlockSpec((k, e), lambda i: (0, 0)),
        ],
        out_specs=[
            pl.BlockSpec((bt, e), cur),
            pl.BlockSpec((bt, 1), cur),
            pl.BlockSpec((bt, k), prev),
            pl.BlockSpec((bt, e), prev),
        ],
        out_shape=[
            jax.ShapeDtypeStruct((b, e), jnp.float32),
            jax.ShapeDtypeStruct((b, 1), jnp.int32),
            jax.ShapeDtypeStruct((b, k), jnp.float32),
            jax.ShapeDtypeStruct((b, e), jnp.float32),
        ],
        scratch_shapes=[pltpu.VMEM((k, e), jnp.float32),
                        pltpu.VMEM((bt, 1), jnp.int32)],
    )(x, w1, b1.reshape(1, m), w2, b2.reshape(1, e), emb)


def _sc_gather(table, idx, n_streams=8):
    k, d = table.shape
    b = idx.shape[0]
    info = plsc.get_sparse_core_info()
    nw = info.num_cores * info.num_subcores
    b_per_w = b // nw
    chunk = b_per_w // n_streams
    idx3 = idx.reshape(nw, n_streams, chunk)
    mesh = plsc.VectorSubcoreMesh(core_axis_name="c", subcore_axis_name="s")

    @functools.partial(
        pl.kernel,
        mesh=mesh,
        out_type=jax.ShapeDtypeStruct((b, d), jnp.float32),
        scratch_types=[
            pltpu.VMEM((n_streams, chunk), jnp.int32),
            pltpu.VMEM((b_per_w, d), jnp.float32),
            pltpu.SemaphoreType.DMA,
        ],
    )
    def gather_kernel(table_hbm, idx_hbm, out_hbm, idx_v, rows_v, sem):
        wid = lax.axis_index("s") * info.num_cores + lax.axis_index("c")
        base = wid * b_per_w
        pltpu.sync_copy(idx_hbm.at[wid], idx_v)
        copies = [
            pltpu.async_copy(table_hbm.at[idx_v.at[j]],
                             rows_v.at[pl.ds(j * chunk, chunk)], sem)
            for j in range(n_streams)
        ]
        for c in copies:
            c.wait()
        pltpu.sync_copy(rows_v, out_hbm.at[pl.ds(base, b_per_w)])

    return gather_kernel(table, idx3)


def _decoder_body(v_ref, w1_ref, b1_ref, w2_ref, b2_ref, out_ref):
    h = jnp.maximum(jnp.dot(v_ref[...], w1_ref[...]) + b1_ref[...], 0.0)
    out_ref[...] = jnp.maximum(jnp.dot(h, w2_ref[...]) + b2_ref[...], 0.0)


def _decoder(v, w1, b1, w2, b2, bt):
    b, e = v.shape
    m = w1.shape[1]
    f = w2.shape[1]
    grid = b // bt
    return pl.pallas_call(
        _decoder_body,
        grid=(grid,),
        in_specs=[
            pl.BlockSpec((bt, e), lambda i: (i, 0)),
            pl.BlockSpec((e, m), lambda i: (0, 0)),
            pl.BlockSpec((1, m), lambda i: (0, 0)),
            pl.BlockSpec((m, f), lambda i: (0, 0)),
            pl.BlockSpec((1, f), lambda i: (0, 0)),
        ],
        out_specs=pl.BlockSpec((bt, f), lambda i: (i, 0)),
        out_shape=jax.ShapeDtypeStruct((b, f), jnp.float32),
    )(v, w1, b1.reshape(1, m), w2, b2.reshape(1, f))


def kernel(inputs, W_enc1, b_enc1, W_enc2, b_enc2, W_dec1, b_dec1, W_dec2, b_dec2, emb):
    k = emb.shape[0]
    b = inputs.shape[0]
    bt_mm = min(512, b)
    bt_sc = min(256, b)
    encoded, ind2d, oh, vqp = _encode_and_nearest(
        inputs, W_enc1, b_enc1, W_enc2, b_enc2, emb, bt=bt_sc)
    vq_feat = _sc_gather(emb, ind2d.reshape(-1))
    decoded = _decoder(vqp, W_dec1, b_dec1, W_dec2, b_dec2, bt=bt_mm)
    return (encoded, vq_feat, oh, decoded, emb)


# speedup vs baseline: 1.0035x; 1.0035x over previous
import functools

import jax
import jax.numpy as jnp
from jax import lax
from jax.experimental import pallas as pl
from jax.experimental.pallas import tpu as pltpu
from jax.experimental.pallas import tpu_sc as plsc


def _enc_score_body(x_ref, w1_ref, b1_ref, w2_ref, b2_ref, emb_ref,
                    enc_ref, ind_ref, oh_ref, vqp_ref, embn_ref, prev_ref):
    i = pl.program_id(0)
    nt = pl.num_programs(0) - 1

    @pl.when(i == 0)
    def _norm_codebook():
        v = emb_ref[...]
        n = jnp.sqrt(jnp.sum(v * v, axis=1, keepdims=True)) + 1e-12
        embn_ref[...] = v / n

    @pl.when(i > 0)
    def _prev_tile():
        ip = prev_ref[...]
        iota = lax.broadcasted_iota(jnp.int32, oh_ref.shape, 1)
        oh = (iota == ip).astype(jnp.float32)
        oh_ref[...] = oh
        vqp_ref[...] = jnp.dot(oh, emb_ref[...])

    @pl.when(i < nt)
    def _cur_tile():
        h = jnp.maximum(jnp.dot(x_ref[...], w1_ref[...]) + b1_ref[...], 0.0)
        a = jnp.dot(h, w2_ref[...]) + b2_ref[...]
        enc_ref[...] = a
        an = a / (jnp.sqrt(jnp.sum(a * a, axis=1, keepdims=True)) + 1e-12)
        d = -jnp.dot(an, embn_ref[...].T)
        ind = jnp.argmin(d, axis=1).astype(jnp.int32).reshape(-1, 1)
        ind_ref[...] = ind
        prev_ref[...] = ind


def _encode_and_nearest(x, w1, b1, w2, b2, emb, bt):
    b, f = x.shape
    m = w1.shape[1]
    e = w2.shape[1]
    k = emb.shape[0]
    nt = b // bt

    def cur(i):
        return (jnp.minimum(i, nt - 1), 0)

    def prev(i):
        return (jnp.maximum(i - 1, 0), 0)

    return pl.pallas_call(
        _enc_score_body,
        grid=(nt + 1,),
        in_specs=[
            pl.BlockSpec((bt, f), cur),
            pl.BlockSpec((f, m), lambda i: (0, 0)),
            pl.BlockSpec((1, m), lambda i: (0, 0)),
            pl.BlockSpec((m, e), lambda i: (0, 0)),
            pl.BlockSpec((1, e), lambda i: (0, 0)),
            pl.BlockSpec((k, e), lambda i: (0, 0)),
        ],
        out_specs=[
            pl.BlockSpec((bt, e), cur),
            pl.BlockSpec((bt, 1), cur),
            pl.BlockSpec((bt, k), prev),
            pl.BlockSpec((bt, e), prev),
        ],
        out_shape=[
            jax.ShapeDtypeStruct((b, e), jnp.float32),
            jax.ShapeDtypeStruct((b, 1), jnp.int32),
            jax.ShapeDtypeStruct((b, k), jnp.float32),
            jax.ShapeDtypeStruct((b, e), jnp.float32),
        ],
        scratch_shapes=[pltpu.VMEM((k, e), jnp.float32),
                        pltpu.VMEM((bt, 1), jnp.int32)],
    )(x, w1, b1.reshape(1, m), w2, b2.reshape(1, e), emb)


def _sc_gather(table, idx, n_streams=8):
    k, d = table.shape
    b = idx.shape[0]
    info = plsc.get_sparse_core_info()
    nw = info.num_cores * info.num_subcores
    b_per_w = b // nw
    chunk = b_per_w // n_streams
    idx3 = idx.reshape(nw, n_streams, chunk)
    mesh = plsc.VectorSubcoreMesh(core_axis_name="c", subcore_axis_name="s")

    @functools.partial(
        pl.kernel,
        mesh=mesh,
        out_type=jax.ShapeDtypeStruct((b, d), jnp.float32),
        scratch_types=[
            pltpu.VMEM((n_streams, chunk), jnp.int32),
            pltpu.VMEM((b_per_w, d), jnp.float32),
            pltpu.SemaphoreType.DMA,
        ],
    )
    def gather_kernel(table_hbm, idx_hbm, out_hbm, idx_v, rows_v, sem):
        wid = lax.axis_index("s") * info.num_cores + lax.axis_index("c")
        base = wid * b_per_w
        pltpu.sync_copy(idx_hbm.at[wid], idx_v)
        copies = [
            pltpu.async_copy(table_hbm.at[idx_v.at[j]],
                             rows_v.at[pl.ds(j * chunk, chunk)], sem)
            for j in range(n_streams)
        ]
        for c in copies:
            c.wait()
        pltpu.sync_copy(rows_v, out_hbm.at[pl.ds(base, b_per_w)])

    return gather_kernel(table, idx3)


def _decoder_body(v_ref, w1_ref, b1_ref, w2_ref, b2_ref, out_ref):
    h = jnp.maximum(jnp.dot(v_ref[...], w1_ref[...]) + b1_ref[...], 0.0)
    out_ref[...] = jnp.maximum(jnp.dot(h, w2_ref[...]) + b2_ref[...], 0.0)


def _decoder(v, w1, b1, w2, b2, bt):
    b, e = v.shape
    m = w1.shape[1]
    f = w2.shape[1]
    grid = b // bt
    return pl.pallas_call(
        _decoder_body,
        grid=(grid,),
        in_specs=[
            pl.BlockSpec((bt, e), lambda i: (i, 0)),
            pl.BlockSpec((e, m), lambda i: (0, 0)),
            pl.BlockSpec((1, m), lambda i: (0, 0)),
            pl.BlockSpec((m, f), lambda i: (0, 0)),
            pl.BlockSpec((1, f), lambda i: (0, 0)),
        ],
        out_specs=pl.BlockSpec((bt, f), lambda i: (i, 0)),
        out_shape=jax.ShapeDtypeStruct((b, f), jnp.float32),
    )(v, w1, b1.reshape(1, m), w2, b2.reshape(1, f))


def kernel(inputs, W_enc1, b_enc1, W_enc2, b_enc2, W_dec1, b_dec1, W_dec2, b_dec2, emb):
    k = emb.shape[0]
    b = inputs.shape[0]
    bt_mm = min(1024, b)
    bt_sc = min(256, b)
    encoded, ind2d, oh, vqp = _encode_and_nearest(
        inputs, W_enc1, b_enc1, W_enc2, b_enc2, emb, bt=bt_sc)
    vq_feat = _sc_gather(emb, ind2d.reshape(-1))
    decoded = _decoder(vqp, W_dec1, b_dec1, W_dec2, b_dec2, bt=bt_mm)
    return (encoded, vq_feat, oh, decoded, emb)
